# Initial kernel scaffold; baseline (speedup 1.0000x reference)
import jax
import jax.numpy as jnp
from jax.experimental import pallas as pl


def kernel(idx, targets, tok_table, pos_table, W, b):
    raise NotImplementedError("write your pallas kernel here")


# speedup vs baseline: 2.0787x; 2.0787x over previous
import functools

import jax
import jax.numpy as jnp
from jax import lax
from jax.experimental import pallas as pl
from jax.experimental.pallas import tpu as pltpu
from jax.experimental.pallas import tpu_sc as plsc

_NC, _NS = 2, 16
_NW = _NC * _NS
_CHUNK = 128


def _sc_gather(idx2d, table):
    num_chunks, chunk = idx2d.shape
    d = table.shape[1]
    cpw = num_chunks // _NW
    mesh = plsc.VectorSubcoreMesh(core_axis_name="c", subcore_axis_name="s")

    @functools.partial(
        pl.kernel,
        mesh=mesh,
        compiler_params=pltpu.CompilerParams(use_tc_tiling_on_sc=False),
        out_type=jax.ShapeDtypeStruct((num_chunks, chunk, d), jnp.float32),
        scratch_types=[
            pltpu.VMEM((cpw, chunk), jnp.int32),
            pltpu.VMEM((cpw, chunk, d), jnp.float32),
            pltpu.SemaphoreType.DMA,
        ],
    )
    def gather_kernel(idx_hbm, table_hbm, out_hbm, idx_v, rows_v, sem):
        wid = lax.axis_index("s") * _NC + lax.axis_index("c")
        base = wid * cpw
        pltpu.sync_copy(idx_hbm.at[pl.ds(base, cpw)], idx_v)
        copies = [
            pltpu.async_copy(table_hbm.at[idx_v.at[j]], rows_v.at[j], sem)
            for j in range(cpw)
        ]
        for c in copies:
            c.wait()
        pltpu.sync_copy(rows_v, out_hbm.at[pl.ds(base, cpw)])

    return gather_kernel(idx2d, table)


def _tc_head(x, pos_tiled, W, b2, targets2, block_rows):
    bt, d = x.shape
    v = W.shape[1]
    steps = bt // block_rows
    inv_n = 1.0 / bt

    def body(x_ref, pos_ref, w_ref, b_ref, t_ref, logits_ref, loss_ref):
        i = pl.program_id(0)
        xp = x_ref[...] + pos_ref[...]
        logits = (
            jnp.dot(xp, w_ref[...], preferred_element_type=jnp.float32) + b_ref[...]
        )
        logits_ref[...] = logits
        rowmax = jnp.max(logits, axis=1, keepdims=True)
        sumexp = jnp.sum(jnp.exp(logits - rowmax), axis=1, keepdims=True)
        lse = rowmax + jnp.log(sumexp)
        colid = lax.broadcasted_iota(jnp.int32, (block_rows, v), 1)
        tmask = colid == t_ref[...]
        tlogit = jnp.sum(
            jnp.where(tmask, logits, 0.0), axis=1, keepdims=True
        )
        partial = jnp.sum(lse - tlogit, axis=0, keepdims=True) * inv_n

        @pl.when(i == 0)
        def _():
            loss_ref[...] = jnp.zeros_like(loss_ref)

        loss_ref[...] += partial

    return pl.pallas_call(
        body,
        grid=(steps,),
        in_specs=[
            pl.BlockSpec((block_rows, d), lambda i: (i, 0)),
            pl.BlockSpec((block_rows, d), lambda i: (0, 0)),
            pl.BlockSpec((d, v), lambda i: (0, 0)),
            pl.BlockSpec((1, v), lambda i: (0, 0)),
            pl.BlockSpec((block_rows, 1), lambda i: (i, 0)),
        ],
        out_specs=[
            pl.BlockSpec((block_rows, v), lambda i: (i, 0)),
            pl.BlockSpec((1, 1), lambda i: (0, 0)),
        ],
        out_shape=[
            jax.ShapeDtypeStruct((bt, v), jnp.float32),
            jax.ShapeDtypeStruct((1, 1), jnp.float32),
        ],
    )(x, pos_tiled, W, b2, targets2)


def kernel(idx, targets, tok_table, pos_table, W, b):
    B, T = idx.shape
    bt = B * T
    d = tok_table.shape[1]
    v = W.shape[1]
    block_rows = 1024

    idx2d = idx.reshape(bt // _CHUNK, _CHUNK).astype(jnp.int32)
    x = _sc_gather(idx2d, tok_table).reshape(bt, d)

    pos_tiled = jnp.tile(pos_table, (block_rows // T, 1))
    b2 = b.reshape(1, v)
    targets2 = targets.reshape(bt, 1).astype(jnp.int32)

    logits_flat, loss_acc = _tc_head(x, pos_tiled, W, b2, targets2, block_rows)
    return logits_flat.reshape(B, T, v), loss_acc[0, 0]
